# baseline (speedup 1.0000x reference)
import functools

import jax
import jax.numpy as jnp
import numpy as np
from jax import lax
from jax.experimental import pallas as pl
from jax.experimental.pallas import tpu as pltpu
from jax.experimental.pallas import tpu_sc as plsc

ROWS = 128
VOCAB = 100000
D = 128
K = 64
L = 16
UF = 25
NOUTER = VOCAB // (L * UF)
NBINS = 2048
NCH = 5
CHV = VOCAB // (L * NCH)
CHW = CHV * L
CCAP = 2048
CEFF = CCAP - L

_NC = 2
_NS = 16
_NW = _NC * _NS
ROWS_PER_W = ROWS // _NW

_TOP = np.uint32(0x80000000)


def _to_sortable(v):
    b = lax.bitcast_convert_type(v, jnp.uint32)
    flip = (jnp.uint32(0) - (b >> 31)) | _TOP
    return b ^ flip


def _from_sortable(u):
    flip = (jnp.uint32(0) - ((u >> 31) ^ jnp.uint32(1))) | _TOP
    return lax.bitcast_convert_type(u ^ flip, jnp.float32)


def _popcnt(mask):
    return plsc.all_reduce_population_count(mask)[0]


def _find_threshold(hist, nbins, k, start_bin=None):
    nv = nbins // L
    lanes = lax.iota(jnp.int32, L)

    def cond(carry):
        found, j, _, _, _, _ = carry
        return jnp.logical_and(found == 0, j < nv)

    def body(carry):
        found, j, b_star, c_gt, h_b, cum_above = carry
        i = nv - 1 - j
        vec = hist[pl.ds(i * L, L)]
        total = jnp.sum(vec)
        cs = jnp.cumsum(vec)
        ssum = total - cs + vec
        c_ge = cum_above + ssum
        crossed = c_ge >= k
        ncross = jnp.sum(crossed.astype(jnp.int32))
        hit = ncross > 0
        lane = ncross - 1
        sel = lanes == lane
        ssum_l = jnp.sum(jnp.where(sel, ssum, 0))
        vec_l = jnp.sum(jnp.where(sel, vec, 0))
        return (jnp.where(hit, 1, found),
                j + 1,
                jnp.where(hit, i * L + lane, b_star),
                jnp.where(hit, cum_above + ssum_l - vec_l, c_gt),
                jnp.where(hit, vec_l, h_b),
                cum_above + total)

    j0 = (jnp.int32(0) if start_bin is None
          else jnp.int32(nv - 1) - start_bin // L)
    _, _, b_star, c_gt, h_b, _ = lax.while_loop(
        cond, body, (jnp.int32(0), j0, jnp.int32(0), jnp.int32(0),
                     jnp.int32(0), jnp.int32(0)))
    return b_star, c_gt, h_b


def _zero_hist(hist, nbins):
    zeros = jnp.zeros((L,), jnp.int32)

    def body(j, _):
        for t in range(8):
            hist[pl.ds((j * 8 + t) * L, L)] = zeros
        return 0

    lax.fori_loop(0, nbins // (8 * L), body, 0)


def _sc_body(logits_hbm, emb_hbm, out_hbm,
             row_v, hist, candu, candi, vals, idxb, idx64, probs, rows2,
             outbuf, sem, sems):
    wid = lax.axis_index("s") * _NC + lax.axis_index("c")
    ones = jnp.ones((L,), jnp.int32)
    lanes = lax.iota(jnp.int32, L)

    first = wid * ROWS_PER_W
    pltpu.async_copy(logits_hbm.at[first], row_v, sems)

    def row_body(r, _):
        row = first + r
        _zero_hist(hist, 1024)
        pltpu.make_async_copy(logits_hbm.at[first], row_v, sems).wait()


        @plsc.parallel_loop(0, VOCAB // L, 1, unroll=UF)
        def h1(i):
            base = i * L
            u = _to_sortable(row_v[pl.ds(base, L)])
            row_v[pl.ds(base, L)] = lax.bitcast_convert_type(u, jnp.float32)
            plsc.addupdate_scatter(hist, [(u >> 22).astype(jnp.int32)], ones)
        b1, cgt1, h1b = _find_threshold(hist, 1024, jnp.int32(K))
        k2 = jnp.int32(K) - cgt1
        b1u = b1.astype(jnp.uint32)
        cnt = cgt1 + h1b
        small = cnt <= CCAP
        nv3 = (cnt + (L - 1)) // L

        def scan_collect_cand():
            @plsc.parallel_loop(0, VOCAB // L, 1, unroll=UF,
                                carry=jnp.full((L,), -1, jnp.int32))
            def p2(i, cm1):
                base = i * L
                u = lax.bitcast_convert_type(row_v[pl.ds(base, L)],
                                             jnp.uint32)
                ge = (u >> 22) >= b1u
                pos = cm1 + jnp.cumsum(ge.astype(jnp.int32))
                plsc.store_scatter(candu, [pos],
                                   lax.bitcast_convert_type(u, jnp.int32),
                                   mask=ge)
                plsc.store_scatter(candi, [pos], lanes + base, mask=ge)
                return cm1 + plsc.all_reduce_population_count(ge)

            @pl.when(r < ROWS_PER_W - 1)
            def _issue_next():
                pltpu.async_copy(logits_hbm.at[row + 1], row_v, sems)

            _zero_hist(hist, NBINS)

            def hb2(i, maxb):
                u = lax.bitcast_convert_type(candu[pl.ds(i * L, L)],
                                             jnp.uint32)
                valid = (i * L + lanes) < cnt
                m = jnp.logical_and(valid, (u >> 22) == b1u)
                bins = ((u >> 11) & jnp.uint32(0x7FF)).astype(jnp.int32)
                plsc.addupdate_scatter(hist, [bins], ones, mask=m)
                return jnp.maximum(maxb, jnp.where(m, bins, -1))

            mb2 = lax.fori_loop(0, nv3, hb2, jnp.full((L,), -1, jnp.int32))
            b2, cgt2, _ = _find_threshold(hist, NBINS, k2,
                                          start_bin=jnp.max(mb2))
            k3 = k2 - cgt2
            pfx12 = (b1u << 11) | b2.astype(jnp.uint32)
            _zero_hist(hist, NBINS)

            def hb3(i, maxb):
                u = lax.bitcast_convert_type(candu[pl.ds(i * L, L)],
                                             jnp.uint32)
                valid = (i * L + lanes) < cnt
                m = jnp.logical_and(valid, (u >> 11) == pfx12)
                bins = (u & jnp.uint32(0x7FF)).astype(jnp.int32)
                plsc.addupdate_scatter(hist, [bins], ones, mask=m)
                return jnp.maximum(maxb, jnp.where(m, bins, -1))

            mb3 = lax.fori_loop(0, nv3, hb3, jnp.full((L,), -1, jnp.int32))
            b3, cgt3, _ = _find_threshold(hist, NBINS, k3,
                                          start_bin=jnp.max(mb3))
            m_eq = k3 - cgt3
            thr = (pfx12 << 11) | b3.astype(jnp.uint32)

            def coll(i, carry):
                ptr, tk = carry
                u = lax.bitcast_convert_type(candu[pl.ds(i * L, L)],
                                             jnp.uint32)
                idv = candi[pl.ds(i * L, L)]
                valid = (i * L + lanes) < cnt
                gt = jnp.logical_and(valid, u > thr)
                eq = jnp.logical_and(valid, u == thr)
                eqc = jnp.cumsum(eq.astype(jnp.int32))
                take = jnp.logical_and(eq, (tk + eqc) <= m_eq)
                msk = jnp.logical_or(gt, take)
                plsc.store_compressed(vals.at[pl.ds(ptr, L)], u, mask=msk)
                plsc.store_compressed(idxb.at[pl.ds(ptr, L)], idv, mask=msk)
                return ptr + _popcnt(msk), tk + _popcnt(take)

            lax.fori_loop(0, nv3, coll, (jnp.int32(0), jnp.int32(0)))

        def scan_collect_full():
            _zero_hist(hist, NBINS)

            @plsc.parallel_loop(0, VOCAB // L, 1, unroll=UF)
            def s2(i):
                u = lax.bitcast_convert_type(row_v[pl.ds(i * L, L)],
                                             jnp.uint32)
                plsc.addupdate_scatter(
                    hist, [((u >> 11) & jnp.uint32(0x7FF)).astype(jnp.int32)],
                    ones, mask=((u >> 22) == b1u))

            b2, cgt2, _ = _find_threshold(hist, NBINS, k2)
            k3 = k2 - cgt2
            pfx12 = (b1u << 11) | b2.astype(jnp.uint32)
            _zero_hist(hist, NBINS)

            @plsc.parallel_loop(0, VOCAB // L, 1, unroll=UF)
            def s3(i):
                u = lax.bitcast_convert_type(row_v[pl.ds(i * L, L)],
                                             jnp.uint32)
                plsc.addupdate_scatter(
                    hist, [(u & jnp.uint32(0x7FF)).astype(jnp.int32)],
                    ones, mask=((u >> 11) == pfx12))

            b3, cgt3, _ = _find_threshold(hist, NBINS, k3)
            m_eq = k3 - cgt3
            thr = (pfx12 << 11) | b3.astype(jnp.uint32)

            @plsc.parallel_loop(0, VOCAB // L, 1, unroll=UF,
                                carry=(jnp.int32(0), jnp.int32(0)))
            def coll(i, carry):
                ptr, tk = carry
                base = i * L
                u = lax.bitcast_convert_type(row_v[pl.ds(base, L)],
                                             jnp.uint32)
                gt = u > thr
                eq = u == thr
                eqc = jnp.cumsum(eq.astype(jnp.int32))
                take = jnp.logical_and(eq, (tk + eqc) <= m_eq)
                msk = jnp.logical_or(gt, take)
                plsc.store_compressed(vals.at[pl.ds(ptr, L)], u, mask=msk)
                plsc.store_compressed(idxb.at[pl.ds(ptr, L)], lanes + base,
                                      mask=msk)
                return ptr + _popcnt(msk), tk + _popcnt(take)

            @pl.when(r < ROWS_PER_W - 1)
            def _issue_next():
                pltpu.async_copy(logits_hbm.at[row + 1], row_v, sems)

        lax.cond(small, scan_collect_cand, scan_collect_full)

        for c in range(K // L):
            idx64[pl.ds(c * L, L)] = idxb[pl.ds(c * L, L)]
        gcp = pltpu.async_copy(emb_hbm.at[idx64], rows2, sem)

        mx = _from_sortable(vals[pl.ds(0, L)])
        for c in range(1, K // L):
            mx = jnp.maximum(mx, _from_sortable(vals[pl.ds(c * L, L)]))
        mxs = jnp.max(mx)
        s = jnp.float32(0.0)
        for c in range(K // L):
            e = jnp.exp(_from_sortable(vals[pl.ds(c * L, L)]) - mxs)
            probs[pl.ds(c * L, L)] = e
            s = s + jnp.sum(e)
        gcp.wait()

        acc = [jnp.zeros((L,), jnp.float32) for _ in range(D // L)]
        for jv in range(K // L):
            pv = probs[pl.ds(jv * L, L)]
            for l in range(L):
                pj = pv[l]
                j = jv * L + l
                for c in range(D // L):
                    acc[c] = acc[c] + pj * rows2[j, pl.ds(c * L, L)]
        s_vec = jnp.broadcast_to(s, (L,))
        for c in range(D // L):
            outbuf[pl.ds(c * L, L)] = acc[c] / s_vec
        pltpu.sync_copy(outbuf, out_hbm.at[row])
        return 0

    lax.fori_loop(0, ROWS_PER_W, row_body, 0)


_sc_call = functools.partial(
    pl.kernel,
    out_type=jax.ShapeDtypeStruct((ROWS, D), jnp.float32),
    mesh=plsc.VectorSubcoreMesh(core_axis_name="c", subcore_axis_name="s"),
    scratch_types=[
        pltpu.VMEM((VOCAB,), jnp.float32),
        pltpu.VMEM((NBINS,), jnp.int32),
        pltpu.VMEM((CCAP + L,), jnp.int32),
        pltpu.VMEM((CCAP + L,), jnp.int32),
        pltpu.VMEM((K + L,), jnp.uint32),
        pltpu.VMEM((K + L,), jnp.int32),
        pltpu.VMEM((K,), jnp.int32),
        pltpu.VMEM((K,), jnp.float32),
        pltpu.VMEM((K, D), jnp.float32),
        pltpu.VMEM((D,), jnp.float32),
        pltpu.SemaphoreType.DMA,
        pltpu.SemaphoreType.DMA,
    ],
    compiler_params=pltpu.CompilerParams(needs_layout_passes=False),
)(_sc_body)


def kernel(logits, prev_inputs, embedding_weight):
    del prev_inputs
    return _sc_call(logits, embedding_weight)


# speedup vs baseline: 1.6424x; 1.6424x over previous
import functools

import jax
import jax.numpy as jnp
import numpy as np
from jax import lax
from jax.experimental import pallas as pl
from jax.experimental.pallas import tpu as pltpu
from jax.experimental.pallas import tpu_sc as plsc

ROWS = 128
VOCAB = 100000
D = 128
K = 64
L = 16
UF = 10
NOUTER = VOCAB // (L * UF)
NBINS = 2048
NCH = 5
CHV = VOCAB // (L * NCH)
CHW = CHV * L
CCAP = 2048
CEFF = CCAP - L

_NC = 2
_NS = 16
_NW = _NC * _NS
ROWS_PER_W = ROWS // _NW

_TOP = np.uint32(0x80000000)


def _to_sortable(v):
    b = lax.bitcast_convert_type(v, jnp.uint32)
    flip = (jnp.uint32(0) - (b >> 31)) | _TOP
    return b ^ flip


def _from_sortable(u):
    flip = (jnp.uint32(0) - ((u >> 31) ^ jnp.uint32(1))) | _TOP
    return lax.bitcast_convert_type(u ^ flip, jnp.float32)


def _popcnt(mask):
    return plsc.all_reduce_population_count(mask)[0]


def _find_threshold(hist, nbins, k, start_bin=None):
    nv = nbins // L
    lanes = lax.iota(jnp.int32, L)

    def cond(carry):
        found, j, _, _, _, _ = carry
        return jnp.logical_and(found == 0, j < nv)

    def body(carry):
        found, j, b_star, c_gt, h_b, cum_above = carry
        i = nv - 1 - j
        vec = hist[pl.ds(i * L, L)]
        total = jnp.sum(vec)
        cs = jnp.cumsum(vec)
        ssum = total - cs + vec
        c_ge = cum_above + ssum
        crossed = c_ge >= k
        ncross = jnp.sum(crossed.astype(jnp.int32))
        hit = ncross > 0
        lane = ncross - 1
        sel = lanes == lane
        ssum_l = jnp.sum(jnp.where(sel, ssum, 0))
        vec_l = jnp.sum(jnp.where(sel, vec, 0))
        return (jnp.where(hit, 1, found),
                j + 1,
                jnp.where(hit, i * L + lane, b_star),
                jnp.where(hit, cum_above + ssum_l - vec_l, c_gt),
                jnp.where(hit, vec_l, h_b),
                cum_above + total)

    j0 = (jnp.int32(0) if start_bin is None
          else jnp.int32(nv - 1) - start_bin // L)
    _, _, b_star, c_gt, h_b, _ = lax.while_loop(
        cond, body, (jnp.int32(0), j0, jnp.int32(0), jnp.int32(0),
                     jnp.int32(0), jnp.int32(0)))
    return b_star, c_gt, h_b


def _zero_hist(hist, nbins):
    zeros = jnp.zeros((L,), jnp.int32)

    def body(j, _):
        for t in range(8):
            hist[pl.ds((j * 8 + t) * L, L)] = zeros
        return 0

    lax.fori_loop(0, nbins // (8 * L), body, 0)


def _sc_body(logits_hbm, emb_hbm, out_hbm,
             row_v, hist, candu, candi, vals, idxb, idx64, probs, rows2,
             outbuf, sem, sems):
    wid = lax.axis_index("s") * _NC + lax.axis_index("c")
    ones = jnp.ones((L,), jnp.int32)
    lanes = lax.iota(jnp.int32, L)

    first = wid * ROWS_PER_W
    pltpu.async_copy(logits_hbm.at[first], row_v, sems)

    def row_body(r, _):
        row = first + r
        _zero_hist(hist, 1024)
        pltpu.make_async_copy(logits_hbm.at[first], row_v, sems).wait()


        @plsc.parallel_loop(0, VOCAB // L, 1, unroll=UF)
        def h1(i):
            base = i * L
            u = _to_sortable(row_v[pl.ds(base, L)])
            row_v[pl.ds(base, L)] = lax.bitcast_convert_type(u, jnp.float32)
            plsc.addupdate_scatter(hist, [(u >> 22).astype(jnp.int32)], ones)
        b1, cgt1, h1b = _find_threshold(hist, 1024, jnp.int32(K))
        k2 = jnp.int32(K) - cgt1
        b1u = b1.astype(jnp.uint32)
        cnt = cgt1 + h1b
        small = cnt <= CCAP
        nv3 = (cnt + (L - 1)) // L

        def scan_collect_cand():
            @plsc.parallel_loop(0, VOCAB // L, 1, unroll=UF,
                                carry=jnp.full((L,), -1, jnp.int32))
            def p2(i, cm1):
                base = i * L
                u = lax.bitcast_convert_type(row_v[pl.ds(base, L)],
                                             jnp.uint32)
                ge = (u >> 22) >= b1u
                pos = cm1 + jnp.cumsum(ge.astype(jnp.int32))
                plsc.store_scatter(candu, [pos],
                                   lax.bitcast_convert_type(u, jnp.int32),
                                   mask=ge)
                plsc.store_scatter(candi, [pos], lanes + base, mask=ge)
                return cm1 + plsc.all_reduce_population_count(ge)

            @pl.when(r < ROWS_PER_W - 1)
            def _issue_next():
                pltpu.async_copy(logits_hbm.at[row + 1], row_v, sems)

            _zero_hist(hist, NBINS)

            def hb2(i, maxb):
                u = lax.bitcast_convert_type(candu[pl.ds(i * L, L)],
                                             jnp.uint32)
                valid = (i * L + lanes) < cnt
                m = jnp.logical_and(valid, (u >> 22) == b1u)
                bins = ((u >> 11) & jnp.uint32(0x7FF)).astype(jnp.int32)
                plsc.addupdate_scatter(hist, [bins], ones, mask=m)
                return jnp.maximum(maxb, jnp.where(m, bins, -1))

            mb2 = lax.fori_loop(0, nv3, hb2, jnp.full((L,), -1, jnp.int32))
            b2, cgt2, _ = _find_threshold(hist, NBINS, k2,
                                          start_bin=jnp.max(mb2))
            k3 = k2 - cgt2
            pfx12 = (b1u << 11) | b2.astype(jnp.uint32)
            _zero_hist(hist, NBINS)

            def hb3(i, maxb):
                u = lax.bitcast_convert_type(candu[pl.ds(i * L, L)],
                                             jnp.uint32)
                valid = (i * L + lanes) < cnt
                m = jnp.logical_and(valid, (u >> 11) == pfx12)
                bins = (u & jnp.uint32(0x7FF)).astype(jnp.int32)
                plsc.addupdate_scatter(hist, [bins], ones, mask=m)
                return jnp.maximum(maxb, jnp.where(m, bins, -1))

            mb3 = lax.fori_loop(0, nv3, hb3, jnp.full((L,), -1, jnp.int32))
            b3, cgt3, _ = _find_threshold(hist, NBINS, k3,
                                          start_bin=jnp.max(mb3))
            m_eq = k3 - cgt3
            thr = (pfx12 << 11) | b3.astype(jnp.uint32)

            def coll(i, carry):
                ptr, tk = carry
                u = lax.bitcast_convert_type(candu[pl.ds(i * L, L)],
                                             jnp.uint32)
                idv = candi[pl.ds(i * L, L)]
                valid = (i * L + lanes) < cnt
                gt = jnp.logical_and(valid, u > thr)
                eq = jnp.logical_and(valid, u == thr)
                eqc = jnp.cumsum(eq.astype(jnp.int32))
                take = jnp.logical_and(eq, (tk + eqc) <= m_eq)
                msk = jnp.logical_or(gt, take)
                plsc.store_compressed(vals.at[pl.ds(ptr, L)], u, mask=msk)
                plsc.store_compressed(idxb.at[pl.ds(ptr, L)], idv, mask=msk)
                return ptr + _popcnt(msk), tk + _popcnt(take)

            lax.fori_loop(0, nv3, coll, (jnp.int32(0), jnp.int32(0)))

        def scan_collect_full():
            _zero_hist(hist, NBINS)

            @plsc.parallel_loop(0, VOCAB // L, 1, unroll=UF)
            def s2(i):
                u = lax.bitcast_convert_type(row_v[pl.ds(i * L, L)],
                                             jnp.uint32)
                plsc.addupdate_scatter(
                    hist, [((u >> 11) & jnp.uint32(0x7FF)).astype(jnp.int32)],
                    ones, mask=((u >> 22) == b1u))

            b2, cgt2, _ = _find_threshold(hist, NBINS, k2)
            k3 = k2 - cgt2
            pfx12 = (b1u << 11) | b2.astype(jnp.uint32)
            _zero_hist(hist, NBINS)

            @plsc.parallel_loop(0, VOCAB // L, 1, unroll=UF)
            def s3(i):
                u = lax.bitcast_convert_type(row_v[pl.ds(i * L, L)],
                                             jnp.uint32)
                plsc.addupdate_scatter(
                    hist, [(u & jnp.uint32(0x7FF)).astype(jnp.int32)],
                    ones, mask=((u >> 11) == pfx12))

            b3, cgt3, _ = _find_threshold(hist, NBINS, k3)
            m_eq = k3 - cgt3
            thr = (pfx12 << 11) | b3.astype(jnp.uint32)

            @plsc.parallel_loop(0, VOCAB // L, 1, unroll=UF,
                                carry=(jnp.int32(0), jnp.int32(0)))
            def coll(i, carry):
                ptr, tk = carry
                base = i * L
                u = lax.bitcast_convert_type(row_v[pl.ds(base, L)],
                                             jnp.uint32)
                gt = u > thr
                eq = u == thr
                eqc = jnp.cumsum(eq.astype(jnp.int32))
                take = jnp.logical_and(eq, (tk + eqc) <= m_eq)
                msk = jnp.logical_or(gt, take)
                plsc.store_compressed(vals.at[pl.ds(ptr, L)], u, mask=msk)
                plsc.store_compressed(idxb.at[pl.ds(ptr, L)], lanes + base,
                                      mask=msk)
                return ptr + _popcnt(msk), tk + _popcnt(take)

            @pl.when(r < ROWS_PER_W - 1)
            def _issue_next():
                pltpu.async_copy(logits_hbm.at[row + 1], row_v, sems)

        lax.cond(small, scan_collect_cand, scan_collect_full)

        for c in range(K // L):
            idx64[pl.ds(c * L, L)] = idxb[pl.ds(c * L, L)]
        gcp = pltpu.async_copy(emb_hbm.at[idx64], rows2, sem)

        mx = _from_sortable(vals[pl.ds(0, L)])
        for c in range(1, K // L):
            mx = jnp.maximum(mx, _from_sortable(vals[pl.ds(c * L, L)]))
        mxs = jnp.max(mx)
        s = jnp.float32(0.0)
        for c in range(K // L):
            e = jnp.exp(_from_sortable(vals[pl.ds(c * L, L)]) - mxs)
            probs[pl.ds(c * L, L)] = e
            s = s + jnp.sum(e)
        gcp.wait()

        acc = [jnp.zeros((L,), jnp.float32) for _ in range(D // L)]
        for jv in range(K // L):
            pv = probs[pl.ds(jv * L, L)]
            for l in range(L):
                pj = pv[l]
                j = jv * L + l
                for c in range(D // L):
                    acc[c] = acc[c] + pj * rows2[j, pl.ds(c * L, L)]
        s_vec = jnp.broadcast_to(s, (L,))
        for c in range(D // L):
            outbuf[pl.ds(c * L, L)] = acc[c] / s_vec
        pltpu.sync_copy(outbuf, out_hbm.at[row])
        return 0

    lax.fori_loop(0, ROWS_PER_W, row_body, 0)


_sc_call = functools.partial(
    pl.kernel,
    out_type=jax.ShapeDtypeStruct((ROWS, D), jnp.float32),
    mesh=plsc.VectorSubcoreMesh(core_axis_name="c", subcore_axis_name="s"),
    scratch_types=[
        pltpu.VMEM((VOCAB,), jnp.float32),
        pltpu.VMEM((NBINS,), jnp.int32),
        pltpu.VMEM((CCAP + L,), jnp.int32),
        pltpu.VMEM((CCAP + L,), jnp.int32),
        pltpu.VMEM((K + L,), jnp.uint32),
        pltpu.VMEM((K + L,), jnp.int32),
        pltpu.VMEM((K,), jnp.int32),
        pltpu.VMEM((K,), jnp.float32),
        pltpu.VMEM((K, D), jnp.float32),
        pltpu.VMEM((D,), jnp.float32),
        pltpu.SemaphoreType.DMA,
        pltpu.SemaphoreType.DMA,
    ],
    compiler_params=pltpu.CompilerParams(needs_layout_passes=False),
)(_sc_body)


def kernel(logits, prev_inputs, embedding_weight):
    del prev_inputs
    return _sc_call(logits, embedding_weight)
